# Initial kernel scaffold; baseline (speedup 1.0000x reference)
import jax
import jax.numpy as jnp
from jax.experimental import pallas as pl


def kernel(x, t, W_loc0, W_loc1, W_loc2, W_loc3, W_time0, W_time1, W_time2):
    raise NotImplementedError("write your pallas kernel here")


# speedup vs baseline: 6.4380x; 6.4380x over previous
import functools

import jax
import jax.numpy as jnp
from jax import lax
from jax.experimental import pallas as pl
from jax.experimental.pallas import tpu as pltpu
from jax.experimental.pallas import tpu_sc as plsc

B, H = 4096, 50
N = B * H
DL, DT = 64, 32
NLOC, NTIME = 4, 3
NC, NS = 2, 16
NW = NC * NS
PER_W = N // NW
C = 128
NCHUNK = PER_W // C


def _body(x0, x1, x2, x3, t0, t1, t2,
          wl0, wl1, wl2, wl3, wt0, wt1, wt2, out,
          xi0, xi1, xi2, xi3, ti0, ti1, ti2,
          r0, r1, r2, r3, s0, s1, s2, ostage, sem):
    wid = lax.axis_index("s") * NC + lax.axis_index("c")
    w_base = wid * PER_W

    xs = [x0, x1, x2, x3]
    ts = [t0, t1, t2]
    xid = [xi0, xi1, xi2, xi3]
    tid = [ti0, ti1, ti2]
    rl = [r0, r1, r2, r3]
    rt = [s0, s1, s2]
    wls = [wl0, wl1, wl2, wl3]
    wts = [wt0, wt1, wt2]

    def chunk(g, _):
        base = w_base + g * C
        for l in range(NLOC):
            pltpu.sync_copy(xs[l].at[pl.ds(base, C)], xid[l])
        for l in range(NTIME):
            pltpu.sync_copy(ts[l].at[pl.ds(base, C)], tid[l])
        cps = [pltpu.async_copy(wls[l].at[xid[l]], rl[l], sem)
               for l in range(NLOC)]
        cps += [pltpu.async_copy(wts[l].at[tid[l]], rt[l], sem)
                for l in range(NTIME)]
        for cp in cps:
            cp.wait()

        def add_row(i, _):
            for j in range(DL // 16):
                sl = pl.ds(j * 16, 16)
                ostage[i, sl] = ((r0[i, sl] + r1[i, sl])
                                 + (r2[i, sl] + r3[i, sl]))
            for j in range(DT // 16):
                sl = pl.ds(j * 16, 16)
                ostage[i, pl.ds(DL + j * 16, 16)] = (
                    (s0[i, sl] + s1[i, sl]) + s2[i, sl])
            return _
        lax.fori_loop(0, C, add_row, None)

        pltpu.sync_copy(ostage, out.at[pl.ds(base, C)])
        return _

    lax.fori_loop(0, NCHUNK, chunk, None)


@jax.jit
def _emb(x0, x1, x2, x3, t0, t1, t2, wl0, wl1, wl2, wl3, wt0, wt1, wt2):
    mesh = plsc.VectorSubcoreMesh(core_axis_name="c", subcore_axis_name="s")
    scratch = (
        [pltpu.VMEM((C,), jnp.int32) for _ in range(NLOC + NTIME)]
        + [pltpu.VMEM((C, DL), jnp.float32) for _ in range(NLOC)]
        + [pltpu.VMEM((C, DT), jnp.float32) for _ in range(NTIME)]
        + [pltpu.VMEM((C, DL + DT), jnp.float32)]
        + [pltpu.SemaphoreType.DMA]
    )
    return pl.kernel(
        _body,
        out_type=jax.ShapeDtypeStruct((N, DL + DT), jnp.float32),
        mesh=mesh,
        scratch_types=scratch,
        compiler_params=pltpu.CompilerParams(use_tc_tiling_on_sc=False),
    )(x0, x1, x2, x3, t0, t1, t2, wl0, wl1, wl2, wl3, wt0, wt1, wt2)


def kernel(x, t, W_loc0, W_loc1, W_loc2, W_loc3, W_time0, W_time1, W_time2):
    xf = x.reshape(N, NLOC).astype(jnp.int32)
    tf = t.reshape(N, NTIME).astype(jnp.int32)
    xl = [xf[:, l] for l in range(NLOC)]
    tl = [tf[:, l] for l in range(NTIME)]
    out = _emb(*xl, *tl, W_loc0, W_loc1, W_loc2, W_loc3,
               W_time0, W_time1, W_time2)
    return out.reshape(B, H, DL + DT)


# speedup vs baseline: 8.7210x; 1.3546x over previous
import functools

import jax
import jax.numpy as jnp
from jax import lax
from jax.experimental import pallas as pl
from jax.experimental.pallas import tpu as pltpu
from jax.experimental.pallas import tpu_sc as plsc

B, H = 4096, 50
N = B * H
DL, DT = 64, 32
NLOC, NTIME = 4, 3
NLEV = NLOC + NTIME
NC, NS = 2, 16
NW = NC * NS
PER_W = N // NW
C = 128
NCHUNK = PER_W // C
NB = 2


def _body(ia, wl0, wl1, wl2, wl3, wt0, wt1, wt2, out,
          ix0, ix1, r00, r01, r02, r03, r10, r11, r12, r13,
          s00, s01, s02, s10, s11, s12, o0, o1,
          gsem0, gsem1, osem0, osem1):
    wid = lax.axis_index("s") * NC + lax.axis_index("c")

    idx = [ix0, ix1]
    rl = [[r00, r01, r02, r03], [r10, r11, r12, r13]]
    rt = [[s00, s01, s02], [s10, s11, s12]]
    ostage = [o0, o1]
    gsem = [gsem0, gsem1]
    osem = [osem0, osem1]
    wls = [wl0, wl1, wl2, wl3]
    wts = [wt0, wt1, wt2]

    def fire(g, b):
        cid = wid * NCHUNK + g
        pltpu.sync_copy(ia.at[cid], idx[b])
        for l in range(NLOC):
            pltpu.async_copy(wls[l].at[idx[b].at[l]], rl[b][l], gsem[b])
        for l in range(NTIME):
            pltpu.async_copy(wts[l].at[idx[b].at[NLOC + l]], rt[b][l],
                             gsem[b])

    def drain_gathers(b):
        for l in range(NLOC):
            pltpu.make_async_copy(wls[l].at[pl.ds(0, C)], rl[b][l],
                                  gsem[b]).wait()
        for l in range(NTIME):
            pltpu.make_async_copy(wts[l].at[pl.ds(0, C)], rt[b][l],
                                  gsem[b]).wait()

    def drain_out(b):
        pltpu.make_async_copy(ostage[b], out.at[pl.ds(0, C)],
                              osem[b]).wait()

    def compute(b):
        r0, r1, r2, r3 = rl[b]
        s0, s1, s2 = rt[b]
        ob = ostage[b]

        def add_row(i, _):
            for j in range(DL // 16):
                sl = pl.ds(j * 16, 16)
                ob[i, sl] = (r0[i, sl] + r1[i, sl]) + (r2[i, sl] + r3[i, sl])
            for j in range(DT // 16):
                sl = pl.ds(j * 16, 16)
                ob[i, pl.ds(DL + j * 16, 16)] = (
                    (s0[i, sl] + s1[i, sl]) + s2[i, sl])
            return _
        lax.fori_loop(0, C, add_row, None)

    for b in range(NB):
        fire(b, b)

    def step(i, _):
        for b in range(NB):
            g = i * NB + b
            drain_gathers(b)

            @pl.when(i > 0)
            def _w():
                drain_out(b)

            compute(b)
            base = (wid * NCHUNK + g) * C
            pltpu.async_copy(ostage[b], out.at[pl.ds(base, C)], osem[b])

            @pl.when(g + NB < NCHUNK)
            def _f():
                fire(g + NB, b)
        return _

    lax.fori_loop(0, NCHUNK // NB, step, None)
    for b in range(NB):
        drain_out(b)


@jax.jit
def _emb(ia, wl0, wl1, wl2, wl3, wt0, wt1, wt2):
    mesh = plsc.VectorSubcoreMesh(core_axis_name="c", subcore_axis_name="s")
    scratch = (
        [pltpu.VMEM((NLEV, C), jnp.int32) for _ in range(NB)]
        + [pltpu.VMEM((C, DL), jnp.float32) for _ in range(NB * NLOC)]
        + [pltpu.VMEM((C, DT), jnp.float32) for _ in range(NB * NTIME)]
        + [pltpu.VMEM((C, DL + DT), jnp.float32) for _ in range(NB)]
        + [pltpu.SemaphoreType.DMA for _ in range(2 * NB)]
    )
    return pl.kernel(
        _body,
        out_type=jax.ShapeDtypeStruct((N, DL + DT), jnp.float32),
        mesh=mesh,
        scratch_types=scratch,
        compiler_params=pltpu.CompilerParams(use_tc_tiling_on_sc=False),
    )(ia, wl0, wl1, wl2, wl3, wt0, wt1, wt2)


def kernel(x, t, W_loc0, W_loc1, W_loc2, W_loc3, W_time0, W_time1, W_time2):
    xf = x.reshape(N, NLOC).astype(jnp.int32)
    tf = t.reshape(N, NTIME).astype(jnp.int32)
    ia = jnp.concatenate([xf, tf], axis=1).reshape(N // C, C, NLEV)
    ia = ia.transpose(0, 2, 1)
    out = _emb(ia, W_loc0, W_loc1, W_loc2, W_loc3,
               W_time0, W_time1, W_time2)
    return out.reshape(B, H, DL + DT)


# speedup vs baseline: 11.2801x; 1.2934x over previous
import functools

import jax
import jax.numpy as jnp
from jax import lax
from jax.experimental import pallas as pl
from jax.experimental.pallas import tpu as pltpu
from jax.experimental.pallas import tpu_sc as plsc

B, H = 4096, 50
N = B * H
DL, DT = 64, 32
NLOC, NTIME = 4, 3
NLEV = NLOC + NTIME
NC, NS = 2, 16
NW = NC * NS
PER_W = N // NW
C = 128
NCHUNK = PER_W // C
NB = 2


def _body(ia, wl0, wl1, wl2, wl3, wt0, wt1, wt2, out,
          ix0, ix1, r00, r01, r02, r03, r10, r11, r12, r13,
          s00, s01, s02, s10, s11, s12, o0, o1,
          gsem0, gsem1, osem0, osem1, isem0, isem1):
    wid = lax.axis_index("s") * NC + lax.axis_index("c")

    idx = [ix0, ix1]
    rl = [[r00, r01, r02, r03], [r10, r11, r12, r13]]
    rt = [[s00, s01, s02], [s10, s11, s12]]
    ostage = [o0, o1]
    gsem = [gsem0, gsem1]
    osem = [osem0, osem1]
    isem = [isem0, isem1]
    wls = [wl0, wl1, wl2, wl3]
    wts = [wt0, wt1, wt2]

    def fire_idx(g, b):
        cid = wid * NCHUNK + g
        pltpu.async_copy(ia.at[cid], idx[b], isem[b])

    def fire_gathers(b):
        pltpu.make_async_copy(ia.at[0], idx[b], isem[b]).wait()
        for l in range(NLOC):
            pltpu.async_copy(wls[l].at[idx[b].at[l]], rl[b][l], gsem[b])
        for l in range(NTIME):
            pltpu.async_copy(wts[l].at[idx[b].at[NLOC + l]], rt[b][l],
                             gsem[b])

    def drain_gathers(b):
        for l in range(NLOC):
            pltpu.make_async_copy(wls[l].at[pl.ds(0, C)], rl[b][l],
                                  gsem[b]).wait()
        for l in range(NTIME):
            pltpu.make_async_copy(wts[l].at[pl.ds(0, C)], rt[b][l],
                                  gsem[b]).wait()

    def drain_out(b):
        pltpu.make_async_copy(ostage[b], out.at[pl.ds(0, C)],
                              osem[b]).wait()

    def compute(b):
        r0, r1, r2, r3 = rl[b]
        s0, s1, s2 = rt[b]
        ob = ostage[b]

        @plsc.parallel_loop(0, C, unroll=2)
        def add_row(i):
            for j in range(DL // 16):
                sl = pl.ds(j * 16, 16)
                ob[i, sl] = (r0[i, sl] + r1[i, sl]) + (r2[i, sl] + r3[i, sl])
            for j in range(DT // 16):
                sl = pl.ds(j * 16, 16)
                ob[i, pl.ds(DL + j * 16, 16)] = (
                    (s0[i, sl] + s1[i, sl]) + s2[i, sl])

    for b in range(NB):
        fire_idx(b, b)
    for b in range(NB):
        fire_gathers(b)

    def step(i, _):
        for b in range(NB):
            g = i * NB + b
            drain_gathers(b)

            @pl.when(g + NB < NCHUNK)
            def _i():
                fire_idx(g + NB, b)

            @pl.when(i > 0)
            def _w():
                drain_out(b)

            compute(b)
            base = (wid * NCHUNK + g) * C
            pltpu.async_copy(ostage[b], out.at[pl.ds(base, C)], osem[b])

            @pl.when(g + NB < NCHUNK)
            def _f():
                fire_gathers(b)
        return _

    lax.fori_loop(0, NCHUNK // NB, step, None)
    for b in range(NB):
        drain_out(b)


@jax.jit
def _emb(ia, wl0, wl1, wl2, wl3, wt0, wt1, wt2):
    mesh = plsc.VectorSubcoreMesh(core_axis_name="c", subcore_axis_name="s")
    scratch = (
        [pltpu.VMEM((NLEV, C), jnp.int32) for _ in range(NB)]
        + [pltpu.VMEM((C, DL), jnp.float32) for _ in range(NB * NLOC)]
        + [pltpu.VMEM((C, DT), jnp.float32) for _ in range(NB * NTIME)]
        + [pltpu.VMEM((C, DL + DT), jnp.float32) for _ in range(NB)]
        + [pltpu.SemaphoreType.DMA for _ in range(3 * NB)]
    )
    return pl.kernel(
        _body,
        out_type=jax.ShapeDtypeStruct((N, DL + DT), jnp.float32),
        mesh=mesh,
        scratch_types=scratch,
        compiler_params=pltpu.CompilerParams(use_tc_tiling_on_sc=False),
    )(ia, wl0, wl1, wl2, wl3, wt0, wt1, wt2)


def kernel(x, t, W_loc0, W_loc1, W_loc2, W_loc3, W_time0, W_time1, W_time2):
    xf = x.reshape(N, NLOC).astype(jnp.int32)
    tf = t.reshape(N, NTIME).astype(jnp.int32)
    ia = jnp.concatenate([xf, tf], axis=1).reshape(N // C, C, NLEV)
    ia = ia.transpose(0, 2, 1)
    out = _emb(ia, W_loc0, W_loc1, W_loc2, W_loc3,
               W_time0, W_time1, W_time2)
    return out.reshape(B, H, DL + DT)
